# baseline (speedup 1.0000x reference)
import jax
import jax.numpy as jnp
from jax.experimental import pallas as pl
from jax.experimental.pallas import tpu as pltpu

B = 2048
D = 768
E = 8
TOP_K = 2


def _fused_moe_kernel(x_ref, rw_ref, rb_ref, w_ref, sb_ref, out_ref, wmat_ref):
    e = pl.program_id(0)

    @pl.when(e == 0)
    def _():
        x = x_ref[...]
        logits = jnp.dot(x, rw_ref[...], preferred_element_type=jnp.float32)
        logits = logits + rb_ref[...][None, :]
        idx = jax.lax.broadcasted_iota(jnp.int32, (B, E), 1)
        neg = jnp.float32(-1.7e38)
        v1 = jnp.max(logits, axis=1, keepdims=True)
        i1 = jnp.min(jnp.where(logits == v1, idx, E), axis=1, keepdims=True)
        masked = jnp.where(idx == i1, neg, logits)
        v2 = jnp.max(masked, axis=1, keepdims=True)
        i2 = jnp.min(jnp.where(masked == v2, idx, E), axis=1, keepdims=True)
        w1 = 1.0 / (1.0 + jnp.exp(v2 - v1))
        w2 = 1.0 - w1
        for ee in range(E):
            wmat_ref[ee] = w1 * (i1 == ee).astype(jnp.float32) + w2 * (
                i2 == ee
            ).astype(jnp.float32)
        wmat = jnp.concatenate(
            [wmat_ref[ee] for ee in range(E)], axis=1
        )
        out_ref[...] = jnp.dot(wmat, sb_ref[...], preferred_element_type=jnp.float32)

    y = jnp.dot(x_ref[...], w_ref[...], preferred_element_type=jnp.float32)
    out_ref[...] += wmat_ref[e] * y


@jax.jit
def kernel(x, root_w, root_b, sons_w, sons_b):
    out = pl.pallas_call(
        _fused_moe_kernel,
        grid=(E,),
        in_specs=[
            pl.BlockSpec((B, D), lambda e: (0, 0)),
            pl.BlockSpec((D, E), lambda e: (0, 0)),
            pl.BlockSpec((E,), lambda e: (0,)),
            pl.BlockSpec((D, D), lambda e: (e, 0)),
            pl.BlockSpec((E, D), lambda e: (0, 0)),
        ],
        out_specs=pl.BlockSpec((B, D), lambda e: (0, 0)),
        out_shape=jax.ShapeDtypeStruct((B, D), jnp.float32),
        scratch_shapes=[pltpu.VMEM((E, B, 1), jnp.float32)],
    )(x, root_w, root_b, sons_w.reshape(E * D, D), sons_b)
    return out[:, None, :]
